# baseline (speedup 1.0000x reference)
import dataclasses
import functools

import jax
import jax.numpy as jnp
from jax import lax
from jax.experimental import pallas as pl
from jax.experimental.pallas import tpu as pltpu
from jax.experimental.pallas import tpu_sc as plsc

_P = 7
_NB = _P * _P
_C = 192
_HW = (128, 64, 32, 16)
_SCALE = (0.25, 0.125, 0.0625, 0.03125)
_NW = 32

_UCOLS = _P * 112


def _prep_body(boxes_ref, io, wo, *, n0, lvl_off):
    bx = boxes_ref[...]
    r_pad = bx.shape[0]
    x1 = bx[:, 0:1]
    y1 = bx[:, 1:2]
    x2 = bx[:, 2:3]
    y2 = bx[:, 3:4]
    r_io = lax.broadcasted_iota(jnp.int32, (r_pad, 1), 0)
    b = (r_io >= n0).astype(jnp.int32)

    area = (x2 - x1) * (y2 - y1)
    s = jnp.sqrt(jnp.maximum(area, 0.0))
    t = jnp.floor(4.0 + jnp.log2(s / 224.0 + 1e-6))
    lvl = (jnp.clip(t, 2.0, 5.0) - 2.0).astype(jnp.int32)

    def sel(vals):
        return jnp.where(lvl == 0, vals[0],
               jnp.where(lvl == 1, vals[1],
               jnp.where(lvl == 2, vals[2], vals[3])))

    scale = sel([jnp.float32(v) for v in _SCALE])
    hwf = sel([jnp.float32(v) for v in _HW])
    hw_i = sel([jnp.int32(v) for v in _HW])
    off_i = sel([jnp.int32(v) for v in lvl_off])

    s_io = lax.broadcasted_iota(jnp.int32, (1, _UCOLS), 1)
    py = s_io // 112
    rem = s_io % 112
    k = rem // 28
    il = (rem % 28) // 14
    jj = rem % 14
    ii = py * 2 + il
    offy = (ii.astype(jnp.float32) + 0.5) * 0.5
    offx = (jj.astype(jnp.float32) + 0.5) * 0.5

    sx1 = x1 * scale
    sy1 = y1 * scale
    sx2 = x2 * scale
    sy2 = y2 * scale
    roi_w = jnp.maximum(sx2 - sx1, 1.0)
    roi_h = jnp.maximum(sy2 - sy1, 1.0)
    bin_w = roi_w * (1.0 / _P)
    bin_h = roi_h * (1.0 / _P)
    yy = sy1 + offy * bin_h
    xx = sx1 + offx * bin_w
    valid = ((yy >= -1.0) & (yy <= hwf) & (xx >= -1.0) & (xx <= hwf))
    y = jnp.maximum(yy, 0.0)
    x = jnp.maximum(xx, 0.0)
    ylf = jnp.floor(y)
    xlf = jnp.floor(x)
    hwm1 = hwf - 1.0
    y_low = jnp.minimum(ylf, hwm1)
    x_low = jnp.minimum(xlf, hwm1)
    y_high = jnp.minimum(y_low + 1.0, hwm1)
    x_high = jnp.minimum(x_low + 1.0, hwm1)
    ly = jnp.where(ylf >= hwm1, 0.0, y - ylf)
    lx = jnp.where(xlf >= hwm1, 0.0, x - xlf)
    hy = 1.0 - ly
    hx = 1.0 - lx
    q = jnp.where(valid, 0.25, 0.0)

    base_lo = off_i + (b * hw_i + y_low.astype(jnp.int32)) * hw_i
    base_hi = off_i + (b * hw_i + y_high.astype(jnp.int32)) * hw_i
    ktop = k < 2
    keven = (k % 2) == 0
    ybase = jnp.where(ktop, base_lo, base_hi)
    xv = jnp.where(keven, x_low, x_high).astype(jnp.int32)
    wy = jnp.where(ktop, hy, ly)
    wx = jnp.where(keven, hx, lx)
    io[...] = ybase + xv
    wo[...] = wy * wx * q


def _table_body(p2, p3, p4, p5, out):
    g = pl.program_id(0)

    @pl.when(g < 32)
    def _():
        out[...] = p2[...].reshape(_C, 1024).T

    @pl.when((g >= 32) & (g < 40))
    def _():
        out[...] = p3[...].reshape(_C, 1024).T

    @pl.when((g >= 40) & (g < 42))
    def _():
        out[...] = p4[...].reshape(_C, 1024).T

    @pl.when(g == 42)
    def _():
        a = p5[...]
        out[0:256, :] = a[0].reshape(_C, 256).T
        out[256:512, :] = a[1].reshape(_C, 256).T


def _build_table(feats):
    def imap(start, nblk, ynum):
        def f(g):
            h = jnp.where((g >= start) & (g < start + nblk), g - start, 0)
            return (h // ynum, 0, h % ynum, 0)
        return f

    in_specs = [
        pl.BlockSpec((1, _C, 8, 128), imap(0, 32, 16)),
        pl.BlockSpec((1, _C, 16, 64), imap(32, 8, 4)),
        pl.BlockSpec((1, _C, 32, 32), imap(40, 2, 1)),
        pl.BlockSpec((2, _C, 16, 16), lambda g: (0, 0, 0, 0)),
    ]
    return pl.pallas_call(
        _table_body,
        grid=(43,),
        in_specs=in_specs,
        out_specs=pl.BlockSpec((1024, _C), lambda g: (g, 0)),
        out_shape=jax.ShapeDtypeStruct((43 * 1024, _C), jnp.float32),
    )(*feats)


def _make_sc_kernel(r_pad, r_real):
    rois_per_w = r_pad // _NW
    units_per_w = rois_per_w * _P
    chunk = rois_per_w * _UCOLS
    mesh = plsc.VectorSubcoreMesh(core_axis_name="c", subcore_axis_name="s")
    cp = pltpu.CompilerParams()
    if "needs_layout_passes" in pltpu.CompilerParams.__dataclass_fields__:
        cp = dataclasses.replace(cp, needs_layout_passes=False)
    if "use_tc_tiling_on_sc" in pltpu.CompilerParams.__dataclass_fields__:
        cp = dataclasses.replace(cp, use_tc_tiling_on_sc=False)

    @functools.partial(
        pl.kernel,
        mesh=mesh,
        compiler_params=cp,
        out_type=jax.ShapeDtypeStruct((r_pad * _P, 8, _C), jnp.float32),
        scratch_types=(
            [pltpu.VMEM((chunk,), jnp.int32),
             pltpu.VMEM((chunk,), jnp.float32)]
            + [pltpu.VMEM((112, _C), jnp.float32) for _ in range(2)]
            + [pltpu.VMEM((8, _C), jnp.float32) for _ in range(2)]
            + [pltpu.SemaphoreType.DMA for _ in range(4)]
        ),
    )
    def sc_k(table, idx, wts, out,
             ti, tw, rb_a, rb_b, ob_a, ob_b, gs_a, gs_b, os_a, os_b):
        wid = lax.axis_index("s") * 2 + lax.axis_index("c")
        base = wid * chunk
        pltpu.sync_copy(idx.at[pl.ds(base, chunk)], ti)
        pltpu.sync_copy(wts.at[pl.ds(base, chunk)], tw)
        ubase = wid * units_per_w
        lane = lax.iota(jnp.int32, 16)
        offs = (lane // 4) * 28 + ((lane // 2) % 2) * 14 + (lane % 2)
        splats = [jnp.full((16, 1), n, jnp.int32) for n in range(16)]
        gdn = lax.GatherDimensionNumbers(
            offset_dims=(), collapsed_slice_dims=(0,), start_index_map=(0,))

        def lane_splat(wv, n):
            return lax.gather(
                wv, splats[n], dimension_numbers=gdn, slice_sizes=(1,),
                mode=lax.GatherScatterMode.PROMISE_IN_BOUNDS)

        def col0_of(u):
            return (u // _P) * _UCOLS + (u % _P) * 112

        def gissue(u, rb, sem):
            pltpu.async_copy(table.at[ti.at[pl.ds(col0_of(u), 112)]], rb, sem)

        def gwait(rb, sem):
            pltpu.make_async_copy(
                table.at[ti.at[pl.ds(0, 112)]], rb, sem).wait()

        def oissue(u, ob, sem):
            pltpu.async_copy(ob, out.at[ubase + u], sem)

        def owait(u, ob, sem):
            pltpu.make_async_copy(ob, out.at[ubase + u], sem).wait()

        def compute(u, rb, ob):
            col0 = col0_of(u)
            for px in range(_P):
                wv = plsc.load_gather(tw, [col0 + 2 * px + offs])
                wsp = [lane_splat(wv, n) for n in range(16)]

                @pl.loop(0, _C // 16, unroll=6)
                def _chunk(ci, px=px, wsp=wsp):
                    cs = pl.ds(ci * 16, 16)
                    acc = None
                    n = 0
                    for k in range(4):
                        for il in range(2):
                            for dg in range(2):
                                v = rb[k * 28 + il * 14 + 2 * px + dg, cs] \
                                    * wsp[n]
                                acc = v if acc is None else acc + v
                                n += 1
                    ob[px, cs] = acc

        gissue(0, rb_a, gs_a)

        @pl.loop(0, units_per_w, step=2)
        def _pair(g):
            gissue(g + 1, rb_b, gs_b)
            gwait(rb_a, gs_a)

            @pl.when(g >= 2)
            def _():
                owait(g - 2, ob_a, os_a)

            compute(g, rb_a, ob_a)
            oissue(g, ob_a, os_a)

            @pl.when(g + 2 < units_per_w)
            def _():
                gissue(g + 2, rb_a, gs_a)

            gwait(rb_b, gs_b)

            @pl.when(g >= 2)
            def _():
                owait(g - 1, ob_b, os_b)

            compute(g + 1, rb_b, ob_b)
            oissue(g + 1, ob_b, os_b)

        owait(units_per_w - 2, ob_a, os_a)
        owait(units_per_w - 1, ob_b, os_b)

    return sc_k


def kernel(feat_p2, feat_p3, feat_p4, feat_p5, boxes0, boxes1):
    feats = (feat_p2, feat_p3, feat_p4, feat_p5)
    table = _build_table(feats)
    lvl_off = []
    acc = 0
    for f in feats:
        lvl_off.append(acc)
        acc += f.shape[0] * f.shape[2] * f.shape[3]

    n0 = boxes0.shape[0]
    r = n0 + boxes1.shape[0]
    r_pad = -(-r // _NW) * _NW
    cboxes = jnp.concatenate(
        [boxes0, boxes1, jnp.zeros((r_pad - r, 4), jnp.float32)], axis=0)

    prep = pl.pallas_call(
        functools.partial(_prep_body, n0=n0, lvl_off=tuple(lvl_off)),
        out_shape=[jax.ShapeDtypeStruct((r_pad, _UCOLS), jnp.int32),
                   jax.ShapeDtypeStruct((r_pad, _UCOLS), jnp.float32)],
    )
    io, wo = prep(cboxes)

    sc_k = _make_sc_kernel(r_pad, r)
    out = sc_k(table, io.reshape(-1), wo.reshape(-1))
    res = out[: r * _P, : _P].reshape(r, _P, _P, _C)
    return jnp.transpose(res, (0, 3, 1, 2))


# speedup vs baseline: 1.0319x; 1.0319x over previous
import dataclasses
import functools

import jax
import jax.numpy as jnp
from jax import lax
from jax.experimental import pallas as pl
from jax.experimental.pallas import tpu as pltpu
from jax.experimental.pallas import tpu_sc as plsc

_P = 7
_NB = _P * _P
_C = 192
_HW = (128, 64, 32, 16)
_SCALE = (0.25, 0.125, 0.0625, 0.03125)
_NW = 32

_UCOLS = _P * 112


def _prep_body(boxes_ref, io, wo, *, n0, lvl_off):
    bx = boxes_ref[...]
    r_pad = bx.shape[0]
    x1 = bx[:, 0:1]
    y1 = bx[:, 1:2]
    x2 = bx[:, 2:3]
    y2 = bx[:, 3:4]
    r_io = lax.broadcasted_iota(jnp.int32, (r_pad, 1), 0)
    b = (r_io >= n0).astype(jnp.int32)

    area = (x2 - x1) * (y2 - y1)
    s = jnp.sqrt(jnp.maximum(area, 0.0))
    t = jnp.floor(4.0 + jnp.log2(s / 224.0 + 1e-6))
    lvl = (jnp.clip(t, 2.0, 5.0) - 2.0).astype(jnp.int32)

    def sel(vals):
        return jnp.where(lvl == 0, vals[0],
               jnp.where(lvl == 1, vals[1],
               jnp.where(lvl == 2, vals[2], vals[3])))

    scale = sel([jnp.float32(v) for v in _SCALE])
    hwf = sel([jnp.float32(v) for v in _HW])
    hw_i = sel([jnp.int32(v) for v in _HW])
    off_i = sel([jnp.int32(v) for v in lvl_off])

    s_io = lax.broadcasted_iota(jnp.int32, (1, _UCOLS), 1)
    py = s_io // 112
    rem = s_io % 112
    k = rem // 28
    il = (rem % 28) // 14
    jj = rem % 14
    ii = py * 2 + il
    offy = (ii.astype(jnp.float32) + 0.5) * 0.5
    offx = (jj.astype(jnp.float32) + 0.5) * 0.5

    sx1 = x1 * scale
    sy1 = y1 * scale
    sx2 = x2 * scale
    sy2 = y2 * scale
    roi_w = jnp.maximum(sx2 - sx1, 1.0)
    roi_h = jnp.maximum(sy2 - sy1, 1.0)
    bin_w = roi_w * (1.0 / _P)
    bin_h = roi_h * (1.0 / _P)
    yy = sy1 + offy * bin_h
    xx = sx1 + offx * bin_w
    valid = ((yy >= -1.0) & (yy <= hwf) & (xx >= -1.0) & (xx <= hwf))
    y = jnp.maximum(yy, 0.0)
    x = jnp.maximum(xx, 0.0)
    ylf = jnp.floor(y)
    xlf = jnp.floor(x)
    hwm1 = hwf - 1.0
    y_low = jnp.minimum(ylf, hwm1)
    x_low = jnp.minimum(xlf, hwm1)
    y_high = jnp.minimum(y_low + 1.0, hwm1)
    x_high = jnp.minimum(x_low + 1.0, hwm1)
    ly = jnp.where(ylf >= hwm1, 0.0, y - ylf)
    lx = jnp.where(xlf >= hwm1, 0.0, x - xlf)
    hy = 1.0 - ly
    hx = 1.0 - lx
    q = jnp.where(valid, 0.25, 0.0)

    base_lo = off_i + (b * hw_i + y_low.astype(jnp.int32)) * hw_i
    base_hi = off_i + (b * hw_i + y_high.astype(jnp.int32)) * hw_i
    ktop = k < 2
    keven = (k % 2) == 0
    ybase = jnp.where(ktop, base_lo, base_hi)
    xv = jnp.where(keven, x_low, x_high).astype(jnp.int32)
    wy = jnp.where(ktop, hy, ly)
    wx = jnp.where(keven, hx, lx)
    io[...] = ybase + xv
    wo[...] = wy * wx * q


def _table_body(p2, p3, p4, p5, out):
    g = pl.program_id(0)

    @pl.when(g < 32)
    def _():
        out[...] = p2[...].reshape(_C, 1024).T

    @pl.when((g >= 32) & (g < 40))
    def _():
        out[...] = p3[...].reshape(_C, 1024).T

    @pl.when((g >= 40) & (g < 42))
    def _():
        out[...] = p4[...].reshape(_C, 1024).T

    @pl.when(g == 42)
    def _():
        a = p5[...]
        out[0:256, :] = a[0].reshape(_C, 256).T
        out[256:512, :] = a[1].reshape(_C, 256).T


def _build_table(feats):
    def imap(start, nblk, ynum):
        def f(g):
            h = jnp.where((g >= start) & (g < start + nblk), g - start, 0)
            return (h // ynum, 0, h % ynum, 0)
        return f

    in_specs = [
        pl.BlockSpec((1, _C, 8, 128), imap(0, 32, 16)),
        pl.BlockSpec((1, _C, 16, 64), imap(32, 8, 4)),
        pl.BlockSpec((1, _C, 32, 32), imap(40, 2, 1)),
        pl.BlockSpec((2, _C, 16, 16), lambda g: (0, 0, 0, 0)),
    ]
    return pl.pallas_call(
        _table_body,
        grid=(43,),
        in_specs=in_specs,
        out_specs=pl.BlockSpec((1024, _C), lambda g: (g, 0)),
        out_shape=jax.ShapeDtypeStruct((43 * 1024, _C), jnp.float32),
    )(*feats)


def _make_sc_kernel(r_pad, r_real):
    rois_per_w = r_pad // _NW
    units_per_w = rois_per_w * _P
    chunk = rois_per_w * _UCOLS
    mesh = plsc.VectorSubcoreMesh(core_axis_name="c", subcore_axis_name="s")
    cp = pltpu.CompilerParams()
    if "needs_layout_passes" in pltpu.CompilerParams.__dataclass_fields__:
        cp = dataclasses.replace(cp, needs_layout_passes=False)
    if "use_tc_tiling_on_sc" in pltpu.CompilerParams.__dataclass_fields__:
        cp = dataclasses.replace(cp, use_tc_tiling_on_sc=False)

    @functools.partial(
        pl.kernel,
        mesh=mesh,
        compiler_params=cp,
        out_type=jax.ShapeDtypeStruct((r_pad * _P, 8, _C), jnp.float32),
        scratch_types=(
            [pltpu.VMEM((chunk,), jnp.int32),
             pltpu.VMEM((chunk,), jnp.float32)]
            + [pltpu.VMEM((112, _C), jnp.float32) for _ in range(2)]
            + [pltpu.VMEM((8, _C), jnp.float32) for _ in range(2)]
            + [pltpu.SemaphoreType.DMA for _ in range(4)]
        ),
    )
    def sc_k(table, idx, wts, out,
             ti, tw, rb_a, rb_b, ob_a, ob_b, gs_a, gs_b, os_a, os_b):
        wid = lax.axis_index("s") * 2 + lax.axis_index("c")
        base = wid * chunk
        pltpu.sync_copy(idx.at[pl.ds(base, chunk)], ti)
        pltpu.sync_copy(wts.at[pl.ds(base, chunk)], tw)
        ubase = wid * units_per_w
        lane = lax.iota(jnp.int32, 16)
        offs = (lane // 4) * 28 + ((lane // 2) % 2) * 14 + (lane % 2)
        splats = [jnp.full((16, 1), n, jnp.int32) for n in range(16)]
        gdn = lax.GatherDimensionNumbers(
            offset_dims=(), collapsed_slice_dims=(0,), start_index_map=(0,))

        def lane_splat(wv, n):
            return lax.gather(
                wv, splats[n], dimension_numbers=gdn, slice_sizes=(1,),
                mode=lax.GatherScatterMode.PROMISE_IN_BOUNDS)

        def col0_of(u):
            return (u // _P) * _UCOLS + (u % _P) * 112

        def gissue(u, rb, sem):
            pltpu.async_copy(table.at[ti.at[pl.ds(col0_of(u), 112)]], rb, sem)

        def gwait(rb, sem):
            pltpu.make_async_copy(
                table.at[ti.at[pl.ds(0, 112)]], rb, sem).wait()

        def oissue(u, ob, sem):
            pltpu.async_copy(ob, out.at[ubase + u], sem)

        def owait(u, ob, sem):
            pltpu.make_async_copy(ob, out.at[ubase + u], sem).wait()

        def compute(u, rb, ob):
            col0 = col0_of(u)
            for px in range(_P):
                wv = plsc.load_gather(tw, [col0 + 2 * px + offs])
                wsp = [lane_splat(wv, n) for n in range(16)]

                @pl.loop(0, _C // 16, unroll=3)
                def _chunk(ci, px=px, wsp=wsp):
                    cs = pl.ds(ci * 16, 16)
                    acc = None
                    n = 0
                    for k in range(4):
                        for il in range(2):
                            for dg in range(2):
                                v = rb[k * 28 + il * 14 + 2 * px + dg, cs] \
                                    * wsp[n]
                                acc = v if acc is None else acc + v
                                n += 1
                    ob[px, cs] = acc

        gissue(0, rb_a, gs_a)

        @pl.loop(0, units_per_w, step=2)
        def _pair(g):
            gissue(g + 1, rb_b, gs_b)
            gwait(rb_a, gs_a)

            @pl.when(g >= 2)
            def _():
                owait(g - 2, ob_a, os_a)

            compute(g, rb_a, ob_a)
            oissue(g, ob_a, os_a)

            @pl.when(g + 2 < units_per_w)
            def _():
                gissue(g + 2, rb_a, gs_a)

            gwait(rb_b, gs_b)

            @pl.when(g >= 2)
            def _():
                owait(g - 1, ob_b, os_b)

            compute(g + 1, rb_b, ob_b)
            oissue(g + 1, ob_b, os_b)

        owait(units_per_w - 2, ob_a, os_a)
        owait(units_per_w - 1, ob_b, os_b)

    return sc_k


def kernel(feat_p2, feat_p3, feat_p4, feat_p5, boxes0, boxes1):
    feats = (feat_p2, feat_p3, feat_p4, feat_p5)
    table = _build_table(feats)
    lvl_off = []
    acc = 0
    for f in feats:
        lvl_off.append(acc)
        acc += f.shape[0] * f.shape[2] * f.shape[3]

    n0 = boxes0.shape[0]
    r = n0 + boxes1.shape[0]
    r_pad = -(-r // _NW) * _NW
    cboxes = jnp.concatenate(
        [boxes0, boxes1, jnp.zeros((r_pad - r, 4), jnp.float32)], axis=0)

    prep = pl.pallas_call(
        functools.partial(_prep_body, n0=n0, lvl_off=tuple(lvl_off)),
        out_shape=[jax.ShapeDtypeStruct((r_pad, _UCOLS), jnp.int32),
                   jax.ShapeDtypeStruct((r_pad, _UCOLS), jnp.float32)],
    )
    io, wo = prep(cboxes)

    sc_k = _make_sc_kernel(r_pad, r)
    out = sc_k(table, io.reshape(-1), wo.reshape(-1))
    res = out[: r * _P, : _P].reshape(r, _P, _P, _C)
    return jnp.transpose(res, (0, 3, 1, 2))
